# baseline (speedup 1.0000x reference)
import functools

import jax
import jax.numpy as jnp
from jax import lax
from jax.experimental import pallas as pl
from jax.experimental.pallas import tpu as pltpu
from jax.experimental.pallas import tpu_sc as plsc

_BATCH = 4096
_DIM = 64
_TJ = 512
_TI = 512
_CH = 8

_K1 = 42
_KS2 = 0x1BD11BF0
_TINY = 1.1754943508222875e-38


def _threefry_bits(p):
    u32 = jnp.uint32
    x0 = jnp.zeros_like(p)
    x1 = p + u32(_K1)

    def rnd(x0, x1, r):
        x0 = x0 + x1
        x1 = (x1 << u32(r)) | (x1 >> u32(32 - r))
        return x0, x1 ^ x0

    rot_a = (13, 15, 26, 6)
    rot_b = (17, 29, 16, 24)
    inj = ((_K1, _KS2 + 1), (_KS2, 2), (0, _K1 + 3), (_K1, _KS2 + 4), (_KS2, 5))
    for g in range(5):
        for r in (rot_a if g % 2 == 0 else rot_b):
            x0, x1 = rnd(x0, x1, r)
        a, b = inj[g]
        if a:
            x0 = x0 + u32(a)
        x1 = x1 + u32(b)
    return x0 ^ x1


def _gumbel(p_u32):
    bits = _threefry_bits(p_u32)
    fb = (bits >> jnp.uint32(9)) | jnp.uint32(0x3F800000)
    f = lax.bitcast_convert_type(fb, jnp.float32) - jnp.float32(1.0)
    un = jnp.maximum(jnp.float32(_TINY),
                     f * jnp.float32(1.0 - _TINY) + jnp.float32(_TINY))
    return -jnp.log(-jnp.log(un))


def _sampler_body(x0_ref, x1t_ref, idx_ref, cross_ref):
    jt = pl.program_id(0)
    x1t = x1t_ref[...]
    sq1 = jnp.sum(x1t * x1t, axis=0, keepdims=True)
    col0 = jt * _TJ

    def row_tile(it, carry):
        r0 = it * _TI
        lhs = x0_ref[pl.ds(r0, _TI), :]
        cross_ref[...] = jnp.dot(lhs, x1t, preferred_element_type=jnp.float32)

        def chunk(c, carry):
            vmax, vidx = carry
            rb = r0 + c * _CH
            x0c = x0_ref[pl.ds(rb, _CH), :]
            sq0 = jnp.sum(x0c * x0c, axis=1, keepdims=True)
            crossc = cross_ref[pl.ds(c * _CH, _CH), :]
            logits = -(sq0 - 2.0 * crossc + sq1)
            rows = rb + lax.broadcasted_iota(jnp.int32, (_CH, _TJ), 0)
            cols = col0 + lax.broadcasted_iota(jnp.int32, (_CH, _TJ), 1)
            p = ((rows << 12) + cols).astype(jnp.uint32)
            val = _gumbel(p) + logits
            upd = val > vmax
            vmax = jnp.where(upd, val, vmax)
            vidx = jnp.where(upd, rows, vidx)
            return vmax, vidx

        return lax.fori_loop(0, _TI // _CH, chunk, carry)

    vmax0 = jnp.full((_CH, _TJ), -jnp.inf, jnp.float32)
    vidx0 = jnp.zeros((_CH, _TJ), jnp.int32)
    vmax, vidx = lax.fori_loop(0, _BATCH // _TI, row_tile, (vmax0, vidx0))
    best = jnp.max(vmax, axis=0, keepdims=True)
    sel = jnp.min(jnp.where(vmax == best, vidx, jnp.int32(1 << 30)), axis=0)
    idx_ref[0, 0, :] = sel


def _sample_idx(x0, x1t, interpret=False):
    idx3 = pl.pallas_call(
        _sampler_body,
        grid=(_BATCH // _TJ,),
        in_specs=[
            pl.BlockSpec((_BATCH, _DIM), lambda j: (0, 0)),
            pl.BlockSpec((_DIM, _TJ), lambda j: (0, j)),
        ],
        out_specs=pl.BlockSpec((1, 1, _TJ), lambda j: (j, 0, 0)),
        out_shape=jax.ShapeDtypeStruct((_BATCH // _TJ, 1, _TJ), jnp.int32),
        scratch_shapes=[pltpu.VMEM((_TI, _TJ), jnp.float32)],
        interpret=interpret,
    )(x0, x1t)
    return idx3.reshape(_BATCH)


def _sc_gather(table, idx):
    info = plsc.get_sparse_core_info()
    nw = info.num_cores * info.num_subcores
    bpw = _BATCH // nw
    mesh = plsc.VectorSubcoreMesh(core_axis_name="c", subcore_axis_name="s")

    @functools.partial(
        pl.kernel, mesh=mesh,
        out_type=jax.ShapeDtypeStruct((_BATCH, _DIM), jnp.float32),
        compiler_params=pltpu.CompilerParams(use_tc_tiling_on_sc=False),
        scratch_types=[
            pltpu.VMEM((bpw,), jnp.int32),
            pltpu.VMEM((bpw, _DIM), jnp.float32),
            pltpu.SemaphoreType.DMA,
        ],
    )
    def k(table_hbm, idx_hbm, out_hbm, idx_v, rows_v, sem):
        wid = lax.axis_index("s") * info.num_cores + lax.axis_index("c")
        base = wid * bpw
        pltpu.sync_copy(idx_hbm.at[pl.ds(base, bpw)], idx_v)
        pltpu.async_copy(table_hbm.at[idx_v], rows_v, sem).wait()
        pltpu.sync_copy(rows_v, out_hbm.at[pl.ds(base, bpw)])

    return k(table, idx)


def kernel(x0, x1):
    idx = _sample_idx(x0, x1.T)
    return _sc_gather(x0, idx)


# speedup vs baseline: 1.3972x; 1.3972x over previous
import functools

import jax
import jax.numpy as jnp
from jax import lax
from jax.experimental import pallas as pl
from jax.experimental.pallas import tpu as pltpu
from jax.experimental.pallas import tpu_sc as plsc

_BATCH = 4096
_DIM = 64
_TJ = 512
_TI = 512
_CH = 16

_K1 = 42
_KS2 = 0x1BD11BF0
_TINY = 1.1754943508222875e-38


def _threefry_bits(p):
    u32 = jnp.uint32
    x0 = jnp.zeros_like(p)
    x1 = p + u32(_K1)

    def rnd(x0, x1, r):
        x0 = x0 + x1
        x1 = (x1 << u32(r)) | (x1 >> u32(32 - r))
        return x0, x1 ^ x0

    rot_a = (13, 15, 26, 6)
    rot_b = (17, 29, 16, 24)
    inj = ((_K1, _KS2 + 1), (_KS2, 2), (0, _K1 + 3), (_K1, _KS2 + 4), (_KS2, 5))
    for g in range(5):
        for r in (rot_a if g % 2 == 0 else rot_b):
            x0, x1 = rnd(x0, x1, r)
        a, b = inj[g]
        if a:
            x0 = x0 + u32(a)
        x1 = x1 + u32(b)
    return x0 ^ x1


def _gumbel(p_u32):
    bits = _threefry_bits(p_u32)
    fb = (bits >> jnp.uint32(9)) | jnp.uint32(0x3F800000)
    f = lax.bitcast_convert_type(fb, jnp.float32) - jnp.float32(1.0)
    un = jnp.maximum(jnp.float32(_TINY),
                     f * jnp.float32(1.0 - _TINY) + jnp.float32(_TINY))
    return -jnp.log(-jnp.log(un))


def _sampler_body(x0_ref, x1t_ref, idx_ref, cross_ref):
    jt = pl.program_id(0)
    x1t = x1t_ref[...]
    sq1 = jnp.sum(x1t * x1t, axis=0, keepdims=True)
    col0 = jt * _TJ

    def row_tile(it, carry):
        r0 = it * _TI
        lhs = x0_ref[pl.ds(r0, _TI), :]
        cross_ref[...] = jnp.dot(lhs, x1t, preferred_element_type=jnp.float32)

        def chunk(c, carry):
            vmax, vidx = carry
            rb = r0 + c * _CH
            x0c = x0_ref[pl.ds(rb, _CH), :]
            sq0 = jnp.sum(x0c * x0c, axis=1, keepdims=True)
            crossc = cross_ref[pl.ds(c * _CH, _CH), :]
            logits = -(sq0 - 2.0 * crossc + sq1)
            rows = rb + lax.broadcasted_iota(jnp.int32, (_CH, _TJ), 0)
            cols = col0 + lax.broadcasted_iota(jnp.int32, (_CH, _TJ), 1)
            p = ((rows << 12) + cols).astype(jnp.uint32)
            val = _gumbel(p) + logits
            upd = val > vmax
            vmax = jnp.where(upd, val, vmax)
            vidx = jnp.where(upd, rows, vidx)
            return vmax, vidx

        return lax.fori_loop(0, _TI // _CH, chunk, carry)

    vmax0 = jnp.full((_CH, _TJ), -jnp.inf, jnp.float32)
    vidx0 = jnp.zeros((_CH, _TJ), jnp.int32)
    vmax, vidx = lax.fori_loop(0, _BATCH // _TI, row_tile, (vmax0, vidx0))
    best = jnp.max(vmax, axis=0, keepdims=True)
    sel = jnp.min(jnp.where(vmax == best, vidx, jnp.int32(1 << 30)), axis=0)
    idx_ref[0, 0, :] = sel


def _sample_idx(x0, x1t, interpret=False):
    idx3 = pl.pallas_call(
        _sampler_body,
        grid=(_BATCH // _TJ,),
        in_specs=[
            pl.BlockSpec((_BATCH, _DIM), lambda j: (0, 0)),
            pl.BlockSpec((_DIM, _TJ), lambda j: (0, j)),
        ],
        out_specs=pl.BlockSpec((1, 1, _TJ), lambda j: (j, 0, 0)),
        out_shape=jax.ShapeDtypeStruct((_BATCH // _TJ, 1, _TJ), jnp.int32),
        scratch_shapes=[pltpu.VMEM((_TI, _TJ), jnp.float32)],
        interpret=interpret,
    )(x0, x1t)
    return idx3.reshape(_BATCH)


def _sc_gather(table, idx):
    info = plsc.get_sparse_core_info()
    nw = info.num_cores * info.num_subcores
    bpw = _BATCH // nw
    mesh = plsc.VectorSubcoreMesh(core_axis_name="c", subcore_axis_name="s")

    @functools.partial(
        pl.kernel, mesh=mesh,
        out_type=jax.ShapeDtypeStruct((_BATCH, _DIM), jnp.float32),
        compiler_params=pltpu.CompilerParams(use_tc_tiling_on_sc=False),
        scratch_types=[
            pltpu.VMEM((bpw,), jnp.int32),
            pltpu.VMEM((bpw, _DIM), jnp.float32),
            pltpu.SemaphoreType.DMA,
        ],
    )
    def k(table_hbm, idx_hbm, out_hbm, idx_v, rows_v, sem):
        wid = lax.axis_index("s") * info.num_cores + lax.axis_index("c")
        base = wid * bpw
        pltpu.sync_copy(idx_hbm.at[pl.ds(base, bpw)], idx_v)
        pltpu.async_copy(table_hbm.at[idx_v], rows_v, sem).wait()
        pltpu.sync_copy(rows_v, out_hbm.at[pl.ds(base, bpw)])

    return k(table, idx)


def kernel(x0, x1):
    idx = _sample_idx(x0, x1.T)
    return _sc_gather(x0, idx)


# speedup vs baseline: 1.4014x; 1.0030x over previous
import functools

import jax
import jax.numpy as jnp
from jax import lax
from jax.experimental import pallas as pl
from jax.experimental.pallas import tpu as pltpu
from jax.experimental.pallas import tpu_sc as plsc

_BATCH = 4096
_DIM = 64
_TJ = 512
_TI = 512
_CH = 32

_K1 = 42
_KS2 = 0x1BD11BF0
_TINY = 1.1754943508222875e-38


def _threefry_bits(p):
    u32 = jnp.uint32
    x0 = jnp.zeros_like(p)
    x1 = p + u32(_K1)

    def rnd(x0, x1, r):
        x0 = x0 + x1
        x1 = (x1 << u32(r)) | (x1 >> u32(32 - r))
        return x0, x1 ^ x0

    rot_a = (13, 15, 26, 6)
    rot_b = (17, 29, 16, 24)
    inj = ((_K1, _KS2 + 1), (_KS2, 2), (0, _K1 + 3), (_K1, _KS2 + 4), (_KS2, 5))
    for g in range(5):
        for r in (rot_a if g % 2 == 0 else rot_b):
            x0, x1 = rnd(x0, x1, r)
        a, b = inj[g]
        if a:
            x0 = x0 + u32(a)
        x1 = x1 + u32(b)
    return x0 ^ x1


def _gumbel(p_u32):
    bits = _threefry_bits(p_u32)
    fb = (bits >> jnp.uint32(9)) | jnp.uint32(0x3F800000)
    f = lax.bitcast_convert_type(fb, jnp.float32) - jnp.float32(1.0)
    un = jnp.maximum(jnp.float32(_TINY),
                     f * jnp.float32(1.0 - _TINY) + jnp.float32(_TINY))
    return -jnp.log(-jnp.log(un))


def _sampler_body(x0_ref, x1t_ref, idx_ref, cross_ref):
    jt = pl.program_id(0)
    x1t = x1t_ref[...]
    sq1 = jnp.sum(x1t * x1t, axis=0, keepdims=True)
    col0 = jt * _TJ

    def row_tile(it, carry):
        r0 = it * _TI
        lhs = x0_ref[pl.ds(r0, _TI), :]
        cross_ref[...] = jnp.dot(lhs, x1t, preferred_element_type=jnp.float32)

        def chunk(c, carry):
            vmax, vidx = carry
            rb = r0 + c * _CH
            x0c = x0_ref[pl.ds(rb, _CH), :]
            sq0 = jnp.sum(x0c * x0c, axis=1, keepdims=True)
            crossc = cross_ref[pl.ds(c * _CH, _CH), :]
            logits = -(sq0 - 2.0 * crossc + sq1)
            rows = rb + lax.broadcasted_iota(jnp.int32, (_CH, _TJ), 0)
            cols = col0 + lax.broadcasted_iota(jnp.int32, (_CH, _TJ), 1)
            p = ((rows << 12) + cols).astype(jnp.uint32)
            val = _gumbel(p) + logits
            upd = val > vmax
            vmax = jnp.where(upd, val, vmax)
            vidx = jnp.where(upd, rows, vidx)
            return vmax, vidx

        return lax.fori_loop(0, _TI // _CH, chunk, carry)

    vmax0 = jnp.full((_CH, _TJ), -jnp.inf, jnp.float32)
    vidx0 = jnp.zeros((_CH, _TJ), jnp.int32)
    vmax, vidx = lax.fori_loop(0, _BATCH // _TI, row_tile, (vmax0, vidx0))
    best = jnp.max(vmax, axis=0, keepdims=True)
    sel = jnp.min(jnp.where(vmax == best, vidx, jnp.int32(1 << 30)), axis=0)
    idx_ref[0, 0, :] = sel


def _sample_idx(x0, x1t, interpret=False):
    idx3 = pl.pallas_call(
        _sampler_body,
        grid=(_BATCH // _TJ,),
        in_specs=[
            pl.BlockSpec((_BATCH, _DIM), lambda j: (0, 0)),
            pl.BlockSpec((_DIM, _TJ), lambda j: (0, j)),
        ],
        out_specs=pl.BlockSpec((1, 1, _TJ), lambda j: (j, 0, 0)),
        out_shape=jax.ShapeDtypeStruct((_BATCH // _TJ, 1, _TJ), jnp.int32),
        scratch_shapes=[pltpu.VMEM((_TI, _TJ), jnp.float32)],
        interpret=interpret,
    )(x0, x1t)
    return idx3.reshape(_BATCH)


def _sc_gather(table, idx):
    info = plsc.get_sparse_core_info()
    nw = info.num_cores * info.num_subcores
    bpw = _BATCH // nw
    mesh = plsc.VectorSubcoreMesh(core_axis_name="c", subcore_axis_name="s")

    @functools.partial(
        pl.kernel, mesh=mesh,
        out_type=jax.ShapeDtypeStruct((_BATCH, _DIM), jnp.float32),
        compiler_params=pltpu.CompilerParams(use_tc_tiling_on_sc=False),
        scratch_types=[
            pltpu.VMEM((bpw,), jnp.int32),
            pltpu.VMEM((bpw, _DIM), jnp.float32),
            pltpu.SemaphoreType.DMA,
        ],
    )
    def k(table_hbm, idx_hbm, out_hbm, idx_v, rows_v, sem):
        wid = lax.axis_index("s") * info.num_cores + lax.axis_index("c")
        base = wid * bpw
        pltpu.sync_copy(idx_hbm.at[pl.ds(base, bpw)], idx_v)
        pltpu.async_copy(table_hbm.at[idx_v], rows_v, sem).wait()
        pltpu.sync_copy(rows_v, out_hbm.at[pl.ds(base, bpw)])

    return k(table, idx)


def kernel(x0, x1):
    idx = _sample_idx(x0, x1.T)
    return _sc_gather(x0, idx)
